# baseline (speedup 1.0000x reference)
import dataclasses

import jax
import jax.numpy as jnp
from jax import lax
from jax.experimental import pallas as pl
from jax.experimental.pallas import tpu as pltpu
from jax.experimental.pallas import tpu_sc as plsc

_N = 1_000_000
_NC = 2
_NS = 16
_L = 16
_NW = _NC * _NS
_CH = 1600
_NBLK = _N // _CH
_UNROLL = 4

_POS_BETA = 0.05
_MARGIN_TAU = 0.1
_NORMALIZER = 10

_SENTINEL = 0xFFFFFFFF


def _compiler_params():
    cp = pltpu.CompilerParams()
    if "needs_layout_passes" in pltpu.CompilerParams.__dataclass_fields__:
        cp = dataclasses.replace(cp, needs_layout_passes=False)
    return cp


def _monotone_key(f):
    ui = plsc.bitcast(f, jnp.int32)
    m = lax.shift_right_arithmetic(ui, jnp.int32(31))
    flip = plsc.bitcast(m | jnp.int32(-0x80000000), jnp.uint32)
    return plsc.bitcast(ui, jnp.uint32) ^ flip


def _sc_pass1():
    nb = 2048
    mesh = plsc.VectorSubcoreMesh(core_axis_name="c", subcore_axis_name="s")

    def body(lp_hbm, mk_hbm, out_hbm, keys_hbm, *hists):
        wid = lax.axis_index("s") * _NC + lax.axis_index("c")

        @pl.loop(0, nb, step=_L)
        def _zero(i):
            for h in hists:
                h[pl.ds(i, _L)] = jnp.zeros((_L,), jnp.int32)

        ones = jnp.ones((_L,), jnp.int32)

        def block(lp_v, mk_v, ko_v):
            @pl.loop(0, _CH, step=_UNROLL * _L)
            def _(i):
                fs = [lp_v[pl.ds(i + uu * _L, _L)]
                      for uu in range(_UNROLL)]
                ms = [mk_v[pl.ds(i + uu * _L, _L)]
                      for uu in range(_UNROLL)]
                bs_ = []
                vals = []
                kss = []
                for uu in range(_UNROLL):
                    key = _monotone_key(fs[uu])
                    valid = ms[uu] == 0
                    bs_.append(lax.shift_right_logical(key,
                                                       jnp.uint32(21)))
                    vals.append(valid)
                    kss.append(jnp.where(valid, key,
                                         jnp.uint32(_SENTINEL)))
                for uu in range(_UNROLL):
                    plsc.addupdate_scatter(
                        hists[uu], [plsc.bitcast(bs_[uu], jnp.int32)],
                        ones, mask=vals[uu])
                    ko_v[pl.ds(i + uu * _L, _L)] = plsc.bitcast(
                        kss[uu], jnp.int32)

        pltpu.emit_pipeline(
            block,
            grid=(_NBLK,),
            in_specs=[pl.BlockSpec((_CH,), lambda i: (i,)),
                      pl.BlockSpec((_CH,), lambda i: (i,))],
            out_specs=[pl.BlockSpec((_CH,), lambda i: (i,))],
            core_axis_name=("c", "s"),
            dimension_semantics=(pltpu.PARALLEL,),
        )(lp_hbm, mk_hbm, keys_hbm)

        @pl.loop(0, nb, step=_L)
        def _sum(i):
            s = pl.ds(i, _L)
            hists[0][s] = ((hists[0][s] + hists[1][s])
                           + (hists[2][s] + hists[3][s]))

        pltpu.sync_copy(hists[0], out_hbm.at[wid])

    return pl.kernel(
        body, mesh=mesh,
        out_type=[jax.ShapeDtypeStruct((_NW, nb), jnp.int32),
                  jax.ShapeDtypeStruct((_N,), jnp.int32)],
        scratch_types=[pltpu.VMEM((nb,), jnp.int32)] * _UNROLL,
        compiler_params=_compiler_params(),
    )


def _sc_pass23(nbits, shift):
    nb = 1 << nbits
    mesh = plsc.VectorSubcoreMesh(core_axis_name="c", subcore_axis_name="s")

    def body(keys_hbm, pfx_hbm, out_hbm, pfx_v, *hists):
        pltpu.sync_copy(pfx_hbm, pfx_v)
        wid = lax.axis_index("s") * _NC + lax.axis_index("c")

        @pl.loop(0, nb, step=_L)
        def _zero(i):
            for h in hists:
                h[pl.ds(i, _L)] = jnp.zeros((_L,), jnp.int32)

        ones = jnp.ones((_L,), jnp.int32)

        def block(k_v):
            pshift = lax.shift_left(pfx_v[...], jnp.int32(nbits))

            @pl.loop(0, _CH, step=_UNROLL * _L)
            def _(i):
                keys = [plsc.bitcast(k_v[pl.ds(i + uu * _L, _L)],
                                     jnp.uint32)
                        for uu in range(_UNROLL)]
                ds_ = []
                vs_ = []
                for uu in range(_UNROLL):
                    r = lax.shift_right_logical(keys[uu],
                                                jnp.uint32(shift))
                    d = plsc.bitcast(r, jnp.int32) - pshift
                    ds_.append(d)
                    vs_.append(plsc.bitcast(d, jnp.uint32)
                               < jnp.uint32(nb))
                for uu in range(_UNROLL):
                    plsc.addupdate_scatter(hists[uu], [ds_[uu]], ones,
                                           mask=vs_[uu])

        pltpu.emit_pipeline(
            block,
            grid=(_NBLK,),
            in_specs=[pl.BlockSpec((_CH,), lambda i: (i,))],
            out_specs=[],
            core_axis_name=("c", "s"),
            dimension_semantics=(pltpu.PARALLEL,),
        )(keys_hbm)

        @pl.loop(0, nb, step=_L)
        def _sum(i):
            s = pl.ds(i, _L)
            hists[0][s] = ((hists[0][s] + hists[1][s])
                           + (hists[2][s] + hists[3][s]))

        pltpu.sync_copy(hists[0], out_hbm.at[wid])

    return pl.kernel(
        body, mesh=mesh,
        out_type=jax.ShapeDtypeStruct((_NW, nb), jnp.int32),
        scratch_types=[pltpu.VMEM((_L,), jnp.int32)]
                      + [pltpu.VMEM((nb,), jnp.int32)] * _UNROLL,
        compiler_params=_compiler_params(),
    )


def _find_bucket(h2, kf):
    r = h2.shape[0]
    i0 = lax.broadcasted_iota(jnp.int32, (128, 128), 0)
    i1 = lax.broadcasted_iota(jnp.int32, (128, 128), 1)
    tri = (i0 <= i1).astype(jnp.float32)
    c_row = jnp.dot(h2, tri, preferred_element_type=jnp.float32)
    r0 = lax.broadcasted_iota(jnp.int32, (r, r), 0)
    r1 = lax.broadcasted_iota(jnp.int32, (r, r), 1)
    strict = (r1 < r0).astype(jnp.float32)
    above = jnp.dot(strict, h2, preferred_element_type=jnp.float32)
    offs = jnp.sum(above, axis=1, keepdims=True)
    cum = c_row + offs
    bt = jnp.sum((cum < kf).astype(jnp.int32))
    j0 = lax.broadcasted_iota(jnp.int32, (r, 128), 0)
    j1 = lax.broadcasted_iota(jnp.int32, (r, 128), 1)
    lin = j0 * 128 + j1
    before = jnp.sum(jnp.where(lin < bt, h2, 0.0))
    return bt, before.astype(jnp.int32)


def _merge(ph_ref):
    nb = ph_ref.shape[1]
    h = ph_ref[...].astype(jnp.float32)
    h3 = h.reshape(_NW, nb // 128, 128)
    return jnp.sum(h3, axis=0)


def _pack_meta(k_rem, nv):
    li = lax.broadcasted_iota(jnp.int32, (1, _L), 1)
    return jnp.where(li == 0, k_rem, jnp.where(li == 1, nv, 0))


def _tc_sel1_body(ph_ref, pfx_ref, meta_ref):
    h2 = _merge(ph_ref)
    nvf = jnp.sum(h2)
    k = (nvf * jnp.float32(_POS_BETA)).astype(jnp.int32)
    bt, before = _find_bucket(h2, k.astype(jnp.float32) + 1.0)
    pfx_ref[...] = jnp.full((1, _L), bt, jnp.int32)
    meta_ref[...] = _pack_meta(k - before, nvf.astype(jnp.int32))


def _tc_sel2_body(ph_ref, pfx_in_ref, meta_in_ref, pfx_ref, meta_ref):
    h2 = _merge(ph_ref)
    k1 = meta_in_ref[0, 0]
    nv = meta_in_ref[0, 1]
    b0 = pfx_in_ref[0, 0]
    bt, before = _find_bucket(h2, k1.astype(jnp.float32) + 1.0)
    pfx2 = lax.shift_left(b0, 11) | bt
    pfx_ref[...] = jnp.full((1, _L), pfx2, jnp.int32)
    meta_ref[...] = _pack_meta(k1 - before, nv)


def _tc_final_body(ph_ref, pfx_in_ref, meta_in_ref, bn_ref, ba_ref):
    h2 = _merge(ph_ref)
    k2 = meta_in_ref[0, 0]
    nv = meta_in_ref[0, 1]
    p01 = pfx_in_ref[0, 0]
    bt, _ = _find_bucket(h2, k2.astype(jnp.float32) + 1.0)
    key = lax.shift_left(p01, 10) | bt
    key11 = jnp.full((1, 1), key, jnp.int32)
    ku = lax.bitcast_convert_type(key11, jnp.uint32)
    u = jnp.where(ku >= jnp.uint32(0x80000000),
                  ku ^ jnp.uint32(0x80000000), ~ku)
    val = lax.bitcast_convert_type(u, jnp.float32)
    bn = val / jnp.float32(_NORMALIZER)
    bn = jnp.where(jnp.full((1, 1), nv, jnp.int32) == 0,
                   jnp.float32(jnp.inf), bn)
    bn_ref[...] = bn
    ba_ref[...] = bn - jnp.float32(_MARGIN_TAU)


_sc_cache = {}


def _sc_kernel(tag):
    if tag not in _sc_cache:
        if tag == 1:
            _sc_cache[tag] = _sc_pass1()
        elif tag == 2:
            _sc_cache[tag] = _sc_pass23(11, 10)
        else:
            _sc_cache[tag] = _sc_pass23(10, 0)
    return _sc_cache[tag]


_tc_sel1 = pl.pallas_call(
    _tc_sel1_body,
    out_shape=[jax.ShapeDtypeStruct((1, _L), jnp.int32),
               jax.ShapeDtypeStruct((1, _L), jnp.int32)],
)
_tc_sel2 = pl.pallas_call(
    _tc_sel2_body,
    out_shape=[jax.ShapeDtypeStruct((1, _L), jnp.int32),
               jax.ShapeDtypeStruct((1, _L), jnp.int32)],
)
_tc_final = pl.pallas_call(
    _tc_final_body,
    out_shape=[jax.ShapeDtypeStruct((1, 1), jnp.float32),
               jax.ShapeDtypeStruct((1, 1), jnp.float32)],
)


def kernel(logps, mask):
    ph1, keys = _sc_kernel(1)(logps, mask)
    pfx1, meta1 = _tc_sel1(ph1)
    ph2 = _sc_kernel(2)(keys, pfx1.reshape(_L))
    pfx2, meta2 = _tc_sel2(ph2, pfx1, meta1)
    ph3 = _sc_kernel(3)(keys, pfx2.reshape(_L))
    bn, ba = _tc_final(ph3, pfx2, meta2)
    return bn.reshape(()), ba.reshape(())


# speedup vs baseline: 1.0701x; 1.0701x over previous
import dataclasses

import jax
import jax.numpy as jnp
from jax import lax
from jax.experimental import pallas as pl
from jax.experimental.pallas import tpu as pltpu
from jax.experimental.pallas import tpu_sc as plsc

_N = 1_000_000
_NPAD = 1 << 20
_PAD = _NPAD - _N
_NC = 2
_NS = 16
_L = 16
_NW = _NC * _NS
_NHALF = _NPAD // 2
_CH1 = 4096
_G1 = _NHALF // _CH1
_CH2 = 2048
_G2 = _NHALF // (2 * _CH2)
_UNROLL = 4

_POS_BETA = 0.05
_MARGIN_TAU = 0.1
_NORMALIZER = 10

_SENTINEL = 0xFFFFFFFF


def _compiler_params():
    cp = pltpu.CompilerParams()
    if "needs_layout_passes" in pltpu.CompilerParams.__dataclass_fields__:
        cp = dataclasses.replace(cp, needs_layout_passes=False)
    return cp


def _monotone_key(f):
    ui = plsc.bitcast(f, jnp.int32)
    m = lax.shift_right_arithmetic(ui, jnp.int32(31))
    flip = plsc.bitcast(m | jnp.int32(-0x80000000), jnp.uint32)
    return plsc.bitcast(ui, jnp.uint32) ^ flip


def _sc_pass1():
    nb = 2048
    mesh = plsc.VectorSubcoreMesh(core_axis_name="c", subcore_axis_name="s")

    def body(lp_hbm, mk_hbm, out_hbm, ka_hbm, kb_hbm, *hists):
        wid = lax.axis_index("s") * _NC + lax.axis_index("c")

        @pl.loop(0, nb, step=_L)
        def _zero(i):
            for h in hists:
                h[pl.ds(i, _L)] = jnp.zeros((_L,), jnp.int32)

        ones = jnp.ones((_L,), jnp.int32)

        def block(lpa_v, lpb_v, mka_v, mkb_v, koa_v, kob_v):
            @pl.loop(0, _CH1, step=_UNROLL * _L)
            def _(i):
                for lp_v, mk_v, ko_v in ((lpa_v, mka_v, koa_v),
                                         (lpb_v, mkb_v, kob_v)):
                    fs = [lp_v[pl.ds(i + uu * _L, _L)]
                          for uu in range(_UNROLL)]
                    ms = [mk_v[pl.ds(i + uu * _L, _L)]
                          for uu in range(_UNROLL)]
                    bs_ = []
                    vals = []
                    kss = []
                    for uu in range(_UNROLL):
                        key = _monotone_key(fs[uu])
                        valid = ms[uu] == 0
                        bs_.append(lax.shift_right_logical(key,
                                                           jnp.uint32(21)))
                        vals.append(valid)
                        kss.append(jnp.where(valid, key,
                                             jnp.uint32(_SENTINEL)))
                    for uu in range(_UNROLL):
                        plsc.addupdate_scatter(
                            hists[uu], [plsc.bitcast(bs_[uu], jnp.int32)],
                            ones, mask=vals[uu])
                        ko_v[pl.ds(i + uu * _L, _L)] = plsc.bitcast(
                            kss[uu], jnp.int32)

        pltpu.emit_pipeline(
            block,
            grid=(_G1,),
            in_specs=[pl.BlockSpec((_CH1,), lambda i: (i,)),
                      pl.BlockSpec((_CH1,), lambda i: (i + _G1,)),
                      pl.BlockSpec((_CH1,), lambda i: (i,)),
                      pl.BlockSpec((_CH1,), lambda i: (i + _G1,)),
                      ],
            out_specs=[pl.BlockSpec((_CH1,), lambda i: (i,)),
                       pl.BlockSpec((_CH1,), lambda i: (i,))],
            core_axis_name=("c", "s"),
            dimension_semantics=(pltpu.PARALLEL,),
        )(lp_hbm, lp_hbm, mk_hbm, mk_hbm, ka_hbm, kb_hbm)

        @pl.loop(0, nb, step=_L)
        def _sum(i):
            s = pl.ds(i, _L)
            hists[0][s] = ((hists[0][s] + hists[1][s])
                           + (hists[2][s] + hists[3][s]))

        pltpu.sync_copy(hists[0], out_hbm.at[wid])

    return pl.kernel(
        body, mesh=mesh,
        out_type=[jax.ShapeDtypeStruct((_NW, nb), jnp.int32),
                  jax.ShapeDtypeStruct((_NHALF,), jnp.int32),
                  jax.ShapeDtypeStruct((_NHALF,), jnp.int32)],
        scratch_types=[pltpu.VMEM((nb,), jnp.int32)] * _UNROLL,
        compiler_params=_compiler_params(),
    )


def _sc_pass23(nbits, shift):
    nb = 1 << nbits
    mesh = plsc.VectorSubcoreMesh(core_axis_name="c", subcore_axis_name="s")

    def body(ka_hbm, kb_hbm, pfx_hbm, out_hbm, pfx_v, *hists):
        pltpu.sync_copy(pfx_hbm, pfx_v)
        wid = lax.axis_index("s") * _NC + lax.axis_index("c")

        @pl.loop(0, nb, step=_L)
        def _zero(i):
            for h in hists:
                h[pl.ds(i, _L)] = jnp.zeros((_L,), jnp.int32)

        ones = jnp.ones((_L,), jnp.int32)

        def block(ka0_v, ka1_v, kb0_v, kb1_v):
            pshift = lax.shift_left(pfx_v[...], jnp.int32(nbits))

            @pl.loop(0, _CH2, step=_UNROLL * _L)
            def _(i):
                for k_v in (ka0_v, ka1_v, kb0_v, kb1_v):
                    keys = [plsc.bitcast(k_v[pl.ds(i + uu * _L, _L)],
                                         jnp.uint32)
                            for uu in range(_UNROLL)]
                    ds_ = []
                    vs_ = []
                    for uu in range(_UNROLL):
                        r = lax.shift_right_logical(keys[uu],
                                                    jnp.uint32(shift))
                        d = plsc.bitcast(r, jnp.int32) - pshift
                        ds_.append(d)
                        vs_.append(plsc.bitcast(d, jnp.uint32)
                                   < jnp.uint32(nb))
                    for uu in range(_UNROLL):
                        plsc.addupdate_scatter(hists[uu], [ds_[uu]], ones,
                                               mask=vs_[uu])

        pltpu.emit_pipeline(
            block,
            grid=(_G2,),
            in_specs=[pl.BlockSpec((_CH2,), lambda i: (i,)),
                      pl.BlockSpec((_CH2,), lambda i: (i + _G2,)),
                      pl.BlockSpec((_CH2,), lambda i: (i,)),
                      pl.BlockSpec((_CH2,), lambda i: (i + _G2,))],
            out_specs=[],
            core_axis_name=("c", "s"),
            dimension_semantics=(pltpu.PARALLEL,),
        )(ka_hbm, ka_hbm, kb_hbm, kb_hbm)

        @pl.loop(0, nb, step=_L)
        def _sum(i):
            s = pl.ds(i, _L)
            hists[0][s] = ((hists[0][s] + hists[1][s])
                           + (hists[2][s] + hists[3][s]))

        pltpu.sync_copy(hists[0], out_hbm.at[wid])

    return pl.kernel(
        body, mesh=mesh,
        out_type=jax.ShapeDtypeStruct((_NW, nb), jnp.int32),
        scratch_types=[pltpu.VMEM((_L,), jnp.int32)]
                      + [pltpu.VMEM((nb,), jnp.int32)] * _UNROLL,
        compiler_params=_compiler_params(),
    )


def _find_bucket(h2, kf):
    r = h2.shape[0]
    i0 = lax.broadcasted_iota(jnp.int32, (128, 128), 0)
    i1 = lax.broadcasted_iota(jnp.int32, (128, 128), 1)
    tri = (i0 <= i1).astype(jnp.float32)
    c_row = jnp.dot(h2, tri, preferred_element_type=jnp.float32)
    r0 = lax.broadcasted_iota(jnp.int32, (r, r), 0)
    r1 = lax.broadcasted_iota(jnp.int32, (r, r), 1)
    strict = (r1 < r0).astype(jnp.float32)
    above = jnp.dot(strict, h2, preferred_element_type=jnp.float32)
    offs = jnp.sum(above, axis=1, keepdims=True)
    cum = c_row + offs
    bt = jnp.sum((cum < kf).astype(jnp.int32))
    j0 = lax.broadcasted_iota(jnp.int32, (r, 128), 0)
    j1 = lax.broadcasted_iota(jnp.int32, (r, 128), 1)
    lin = j0 * 128 + j1
    before = jnp.sum(jnp.where(lin < bt, h2, 0.0))
    return bt, before.astype(jnp.int32)


def _merge(ph_ref):
    nb = ph_ref.shape[1]
    h = ph_ref[...].astype(jnp.float32)
    h3 = h.reshape(_NW, nb // 128, 128)
    return jnp.sum(h3, axis=0)


def _pack_meta(k_rem, nv):
    li = lax.broadcasted_iota(jnp.int32, (1, _L), 1)
    return jnp.where(li == 0, k_rem, jnp.where(li == 1, nv, 0))


def _tc_sel1_body(ph_ref, pfx_ref, meta_ref):
    h2 = _merge(ph_ref)
    nvf = jnp.sum(h2)
    k = (nvf * jnp.float32(_POS_BETA)).astype(jnp.int32)
    bt, before = _find_bucket(h2, k.astype(jnp.float32) + 1.0)
    pfx_ref[...] = jnp.full((1, _L), bt, jnp.int32)
    meta_ref[...] = _pack_meta(k - before, nvf.astype(jnp.int32))


def _tc_sel2_body(ph_ref, pfx_in_ref, meta_in_ref, pfx_ref, meta_ref):
    h2 = _merge(ph_ref)
    k1 = meta_in_ref[0, 0]
    nv = meta_in_ref[0, 1]
    b0 = pfx_in_ref[0, 0]
    bt, before = _find_bucket(h2, k1.astype(jnp.float32) + 1.0)
    pfx2 = lax.shift_left(b0, 11) | bt
    pfx_ref[...] = jnp.full((1, _L), pfx2, jnp.int32)
    meta_ref[...] = _pack_meta(k1 - before, nv)


def _tc_final_body(ph_ref, pfx_in_ref, meta_in_ref, bn_ref, ba_ref):
    h2 = _merge(ph_ref)
    k2 = meta_in_ref[0, 0]
    nv = meta_in_ref[0, 1]
    p01 = pfx_in_ref[0, 0]
    bt, _ = _find_bucket(h2, k2.astype(jnp.float32) + 1.0)
    key = lax.shift_left(p01, 10) | bt
    key11 = jnp.full((1, 1), key, jnp.int32)
    ku = lax.bitcast_convert_type(key11, jnp.uint32)
    u = jnp.where(ku >= jnp.uint32(0x80000000),
                  ku ^ jnp.uint32(0x80000000), ~ku)
    val = lax.bitcast_convert_type(u, jnp.float32)
    bn = val / jnp.float32(_NORMALIZER)
    bn = jnp.where(jnp.full((1, 1), nv, jnp.int32) == 0,
                   jnp.float32(jnp.inf), bn)
    bn_ref[...] = bn
    ba_ref[...] = bn - jnp.float32(_MARGIN_TAU)


_sc_cache = {}


def _sc_kernel(tag):
    if tag not in _sc_cache:
        if tag == 1:
            _sc_cache[tag] = _sc_pass1()
        elif tag == 2:
            _sc_cache[tag] = _sc_pass23(11, 10)
        else:
            _sc_cache[tag] = _sc_pass23(10, 0)
    return _sc_cache[tag]


_tc_sel1 = pl.pallas_call(
    _tc_sel1_body,
    out_shape=[jax.ShapeDtypeStruct((1, _L), jnp.int32),
               jax.ShapeDtypeStruct((1, _L), jnp.int32)],
)
_tc_sel2 = pl.pallas_call(
    _tc_sel2_body,
    out_shape=[jax.ShapeDtypeStruct((1, _L), jnp.int32),
               jax.ShapeDtypeStruct((1, _L), jnp.int32)],
)
_tc_final = pl.pallas_call(
    _tc_final_body,
    out_shape=[jax.ShapeDtypeStruct((1, 1), jnp.float32),
               jax.ShapeDtypeStruct((1, 1), jnp.float32)],
)


def kernel(logps, mask):
    lp = jnp.concatenate([logps, jnp.zeros((_PAD,), jnp.float32)])
    mk = jnp.concatenate([mask, jnp.ones((_PAD,), jnp.int32)])
    ph1, ka, kb = _sc_kernel(1)(lp, mk)
    pfx1, meta1 = _tc_sel1(ph1)
    ph2 = _sc_kernel(2)(ka, kb, pfx1.reshape(_L))
    pfx2, meta2 = _tc_sel2(ph2, pfx1, meta1)
    ph3 = _sc_kernel(3)(ka, kb, pfx2.reshape(_L))
    bn, ba = _tc_final(ph3, pfx2, meta2)
    return bn.reshape(()), ba.reshape(())


# speedup vs baseline: 1.1450x; 1.0701x over previous
import dataclasses

import jax
import jax.numpy as jnp
from jax import lax
from jax.experimental import pallas as pl
from jax.experimental.pallas import tpu as pltpu
from jax.experimental.pallas import tpu_sc as plsc

_N = 1_000_000
_NPAD = 1 << 20
_PAD = _NPAD - _N
_NC = 2
_NS = 16
_L = 16
_NW = _NC * _NS
_NHALF = _NPAD // 2
_CH1 = 4096
_G1 = _NHALF // _CH1
_CH2 = 2048
_G2 = _NHALF // (2 * _CH2)
_UNROLL = 4

_POS_BETA = 0.05
_MARGIN_TAU = 0.1
_NORMALIZER = 10

_SENTINEL = 0xFFFFFFFF


def _compiler_params():
    cp = pltpu.CompilerParams()
    if "needs_layout_passes" in pltpu.CompilerParams.__dataclass_fields__:
        cp = dataclasses.replace(cp, needs_layout_passes=False)
    return cp


def _monotone_key(f):
    ui = plsc.bitcast(f, jnp.int32)
    m = lax.shift_right_arithmetic(ui, jnp.int32(31))
    flip = plsc.bitcast(m | jnp.int32(-0x80000000), jnp.uint32)
    return plsc.bitcast(ui, jnp.uint32) ^ flip


def _sc_pass1():
    nb = 2048
    mesh = plsc.VectorSubcoreMesh(core_axis_name="c", subcore_axis_name="s")

    def body(lp_hbm, mk_hbm, out_hbm, ka_hbm, kb_hbm, *hists):
        wid = lax.axis_index("s") * _NC + lax.axis_index("c")

        @pl.loop(0, nb, step=_L)
        def _zero(i):
            for h in hists:
                h[pl.ds(i, _L)] = jnp.zeros((_L,), jnp.int32)

        ones = jnp.ones((_L,), jnp.int32)

        def block(lpa_v, lpb_v, mka_v, mkb_v, koa_v, kob_v):
            @pl.loop(0, _CH1, step=_UNROLL * _L)
            def _(i):
                for lp_v, mk_v, ko_v in ((lpa_v, mka_v, koa_v),
                                         (lpb_v, mkb_v, kob_v)):
                    fs = [lp_v[pl.ds(i + uu * _L, _L)]
                          for uu in range(_UNROLL)]
                    ms = [mk_v[pl.ds(i + uu * _L, _L)]
                          for uu in range(_UNROLL)]
                    bs_ = []
                    vals = []
                    kss = []
                    for uu in range(_UNROLL):
                        key = _monotone_key(fs[uu])
                        valid = ms[uu] == 0
                        bs_.append(lax.shift_right_logical(key,
                                                           jnp.uint32(21)))
                        vals.append(valid)
                        kss.append(jnp.where(valid, key,
                                             jnp.uint32(_SENTINEL)))
                    for uu in range(_UNROLL):
                        plsc.addupdate_scatter(
                            hists[uu], [plsc.bitcast(bs_[uu], jnp.int32)],
                            ones, mask=vals[uu])
                        ko_v[pl.ds(i + uu * _L, _L)] = plsc.bitcast(
                            kss[uu], jnp.int32)

        pltpu.emit_pipeline(
            block,
            grid=(_G1,),
            in_specs=[pl.BlockSpec((_CH1,), lambda i: (i,)),
                      pl.BlockSpec((_CH1,), lambda i: (i + _G1,)),
                      pl.BlockSpec((_CH1,), lambda i: (i,)),
                      pl.BlockSpec((_CH1,), lambda i: (i + _G1,)),
                      ],
            out_specs=[pl.BlockSpec((_CH1,), lambda i: (i,)),
                       pl.BlockSpec((_CH1,), lambda i: (i,))],
            core_axis_name=("c", "s"),
            dimension_semantics=(pltpu.PARALLEL,),
        )(lp_hbm, lp_hbm, mk_hbm, mk_hbm, ka_hbm, kb_hbm)

        @pl.loop(0, nb, step=_L)
        def _sum(i):
            s = pl.ds(i, _L)
            hists[0][s] = ((hists[0][s] + hists[1][s])
                           + (hists[2][s] + hists[3][s]))

        pltpu.sync_copy(hists[0], out_hbm.at[wid])

    return pl.kernel(
        body, mesh=mesh,
        out_type=[jax.ShapeDtypeStruct((_NW, nb), jnp.int32),
                  jax.ShapeDtypeStruct((_NHALF,), jnp.int32),
                  jax.ShapeDtypeStruct((_NHALF,), jnp.int32)],
        scratch_types=[pltpu.VMEM((nb,), jnp.int32)] * _UNROLL,
        compiler_params=_compiler_params(),
    )


def _sc_pass23(nbits, shift):
    nb = 1 << nbits
    mesh = plsc.VectorSubcoreMesh(core_axis_name="c", subcore_axis_name="s")

    def body(ka_hbm, kb_hbm, pfx_hbm, out_hbm, pfx_v, *hists):
        pltpu.sync_copy(pfx_hbm, pfx_v)
        wid = lax.axis_index("s") * _NC + lax.axis_index("c")

        @pl.loop(0, nb, step=_L)
        def _zero(i):
            for h in hists:
                h[pl.ds(i, _L)] = jnp.zeros((_L,), jnp.int32)

        ones = jnp.ones((_L,), jnp.int32)

        def block(ka0_v, ka1_v, kb0_v, kb1_v):
            pshift = lax.shift_left(pfx_v[...], jnp.int32(nbits))
            krefs = (ka0_v, ka1_v, kb0_v, kb1_v)

            @pl.loop(0, _CH2, step=2 * _L)
            def _(i):
                keys = [plsc.bitcast(k_v[pl.ds(i + uu * _L, _L)],
                                     jnp.uint32)
                        for k_v in krefs for uu in range(2)]
                ds_ = []
                vs_ = []
                for uu in range(8):
                    r = lax.shift_right_logical(keys[uu],
                                                jnp.uint32(shift))
                    d = plsc.bitcast(r, jnp.int32) - pshift
                    ds_.append(d)
                    vs_.append(plsc.bitcast(d, jnp.uint32)
                               < jnp.uint32(nb))
                for uu in range(8):
                    plsc.addupdate_scatter(hists[uu % _UNROLL], [ds_[uu]],
                                           ones, mask=vs_[uu])

        pltpu.emit_pipeline(
            block,
            grid=(_G2,),
            in_specs=[pl.BlockSpec((_CH2,), lambda i: (i,)),
                      pl.BlockSpec((_CH2,), lambda i: (i + _G2,)),
                      pl.BlockSpec((_CH2,), lambda i: (i,)),
                      pl.BlockSpec((_CH2,), lambda i: (i + _G2,))],
            out_specs=[],
            core_axis_name=("c", "s"),
            dimension_semantics=(pltpu.PARALLEL,),
        )(ka_hbm, ka_hbm, kb_hbm, kb_hbm)

        @pl.loop(0, nb, step=_L)
        def _sum(i):
            s = pl.ds(i, _L)
            hists[0][s] = ((hists[0][s] + hists[1][s])
                           + (hists[2][s] + hists[3][s]))

        pltpu.sync_copy(hists[0], out_hbm.at[wid])

    return pl.kernel(
        body, mesh=mesh,
        out_type=jax.ShapeDtypeStruct((_NW, nb), jnp.int32),
        scratch_types=[pltpu.VMEM((_L,), jnp.int32)]
                      + [pltpu.VMEM((nb,), jnp.int32)] * _UNROLL,
        compiler_params=_compiler_params(),
    )


def _find_bucket(h2, kf):
    r = h2.shape[0]
    i0 = lax.broadcasted_iota(jnp.int32, (128, 128), 0)
    i1 = lax.broadcasted_iota(jnp.int32, (128, 128), 1)
    tri = (i0 <= i1).astype(jnp.float32)
    c_row = jnp.dot(h2, tri, preferred_element_type=jnp.float32)
    r0 = lax.broadcasted_iota(jnp.int32, (r, r), 0)
    r1 = lax.broadcasted_iota(jnp.int32, (r, r), 1)
    strict = (r1 < r0).astype(jnp.float32)
    above = jnp.dot(strict, h2, preferred_element_type=jnp.float32)
    offs = jnp.sum(above, axis=1, keepdims=True)
    cum = c_row + offs
    bt = jnp.sum((cum < kf).astype(jnp.int32))
    j0 = lax.broadcasted_iota(jnp.int32, (r, 128), 0)
    j1 = lax.broadcasted_iota(jnp.int32, (r, 128), 1)
    lin = j0 * 128 + j1
    before = jnp.sum(jnp.where(lin < bt, h2, 0.0))
    return bt, before.astype(jnp.int32)


def _merge(ph_ref):
    nb = ph_ref.shape[1]
    h = ph_ref[...].astype(jnp.float32)
    h3 = h.reshape(_NW, nb // 128, 128)
    return jnp.sum(h3, axis=0)


def _pack_meta(k_rem, nv):
    li = lax.broadcasted_iota(jnp.int32, (1, _L), 1)
    return jnp.where(li == 0, k_rem, jnp.where(li == 1, nv, 0))


def _tc_sel1_body(ph_ref, pfx_ref, meta_ref):
    h2 = _merge(ph_ref)
    nvf = jnp.sum(h2)
    k = (nvf * jnp.float32(_POS_BETA)).astype(jnp.int32)
    bt, before = _find_bucket(h2, k.astype(jnp.float32) + 1.0)
    pfx_ref[...] = jnp.full((1, _L), bt, jnp.int32)
    meta_ref[...] = _pack_meta(k - before, nvf.astype(jnp.int32))


def _tc_sel2_body(ph_ref, pfx_in_ref, meta_in_ref, pfx_ref, meta_ref):
    h2 = _merge(ph_ref)
    k1 = meta_in_ref[0, 0]
    nv = meta_in_ref[0, 1]
    b0 = pfx_in_ref[0, 0]
    bt, before = _find_bucket(h2, k1.astype(jnp.float32) + 1.0)
    pfx2 = lax.shift_left(b0, 11) | bt
    pfx_ref[...] = jnp.full((1, _L), pfx2, jnp.int32)
    meta_ref[...] = _pack_meta(k1 - before, nv)


def _tc_final_body(ph_ref, pfx_in_ref, meta_in_ref, bn_ref, ba_ref):
    h2 = _merge(ph_ref)
    k2 = meta_in_ref[0, 0]
    nv = meta_in_ref[0, 1]
    p01 = pfx_in_ref[0, 0]
    bt, _ = _find_bucket(h2, k2.astype(jnp.float32) + 1.0)
    key = lax.shift_left(p01, 10) | bt
    key11 = jnp.full((1, 1), key, jnp.int32)
    ku = lax.bitcast_convert_type(key11, jnp.uint32)
    u = jnp.where(ku >= jnp.uint32(0x80000000),
                  ku ^ jnp.uint32(0x80000000), ~ku)
    val = lax.bitcast_convert_type(u, jnp.float32)
    bn = val / jnp.float32(_NORMALIZER)
    bn = jnp.where(jnp.full((1, 1), nv, jnp.int32) == 0,
                   jnp.float32(jnp.inf), bn)
    bn_ref[...] = bn
    ba_ref[...] = bn - jnp.float32(_MARGIN_TAU)


_sc_cache = {}


def _sc_kernel(tag):
    if tag not in _sc_cache:
        if tag == 1:
            _sc_cache[tag] = _sc_pass1()
        elif tag == 2:
            _sc_cache[tag] = _sc_pass23(11, 10)
        else:
            _sc_cache[tag] = _sc_pass23(10, 0)
    return _sc_cache[tag]


_tc_sel1 = pl.pallas_call(
    _tc_sel1_body,
    out_shape=[jax.ShapeDtypeStruct((1, _L), jnp.int32),
               jax.ShapeDtypeStruct((1, _L), jnp.int32)],
)
_tc_sel2 = pl.pallas_call(
    _tc_sel2_body,
    out_shape=[jax.ShapeDtypeStruct((1, _L), jnp.int32),
               jax.ShapeDtypeStruct((1, _L), jnp.int32)],
)
_tc_final = pl.pallas_call(
    _tc_final_body,
    out_shape=[jax.ShapeDtypeStruct((1, 1), jnp.float32),
               jax.ShapeDtypeStruct((1, 1), jnp.float32)],
)


def kernel(logps, mask):
    lp = jnp.concatenate([logps, jnp.zeros((_PAD,), jnp.float32)])
    mk = jnp.concatenate([mask, jnp.ones((_PAD,), jnp.int32)])
    ph1, ka, kb = _sc_kernel(1)(lp, mk)
    pfx1, meta1 = _tc_sel1(ph1)
    ph2 = _sc_kernel(2)(ka, kb, pfx1.reshape(_L))
    pfx2, meta2 = _tc_sel2(ph2, pfx1, meta1)
    ph3 = _sc_kernel(3)(ka, kb, pfx2.reshape(_L))
    bn, ba = _tc_final(ph3, pfx2, meta2)
    return bn.reshape(()), ba.reshape(())


# speedup vs baseline: 1.1663x; 1.0186x over previous
import dataclasses

import jax
import jax.numpy as jnp
from jax import lax
from jax.experimental import pallas as pl
from jax.experimental.pallas import tpu as pltpu
from jax.experimental.pallas import tpu_sc as plsc

_N = 1_000_000
_NPAD = 1 << 20
_PAD = _NPAD - _N
_NC = 2
_NS = 16
_L = 16
_NW = _NC * _NS
_NHALF = _NPAD // 2
_CH1 = 4096
_G1 = _NHALF // _CH1
_CH2 = 2048
_G2 = _NHALF // (2 * _CH2)
_UNROLL = 4

_POS_BETA = 0.05
_MARGIN_TAU = 0.1
_NORMALIZER = 10

_SENTINEL = 0xFFFFFFFF


def _compiler_params():
    cp = pltpu.CompilerParams()
    if "needs_layout_passes" in pltpu.CompilerParams.__dataclass_fields__:
        cp = dataclasses.replace(cp, needs_layout_passes=False)
    return cp


def _monotone_key(f):
    ui = plsc.bitcast(f, jnp.int32)
    m = lax.shift_right_arithmetic(ui, jnp.int32(31))
    flip = plsc.bitcast(m | jnp.int32(-0x80000000), jnp.uint32)
    return plsc.bitcast(ui, jnp.uint32) ^ flip


def _sc_pass1():
    nb = 2048
    mesh = plsc.VectorSubcoreMesh(core_axis_name="c", subcore_axis_name="s")

    def body(lp_hbm, mk_hbm, out_hbm, ka_hbm, kb_hbm, *hists):
        wid = lax.axis_index("s") * _NC + lax.axis_index("c")

        @pl.loop(0, nb, step=_L)
        def _zero(i):
            for h in hists:
                h[pl.ds(i, _L)] = jnp.zeros((_L,), jnp.int32)

        ones = jnp.ones((_L,), jnp.int32)

        def block(lpa_v, lpb_v, mka_v, mkb_v, koa_v, kob_v):
            halves = ((lpa_v, mka_v, koa_v), (lpb_v, mkb_v, kob_v))

            @pl.loop(0, _CH1, step=_UNROLL * _L)
            def _(i):
                fs = [lp_v[pl.ds(i + uu * _L, _L)]
                      for lp_v, _m, _k in halves for uu in range(_UNROLL)]
                ms = [mk_v[pl.ds(i + uu * _L, _L)]
                      for _l, mk_v, _k in halves for uu in range(_UNROLL)]
                bs_ = []
                vals = []
                kss = []
                for uu in range(2 * _UNROLL):
                    key = _monotone_key(fs[uu])
                    valid = ms[uu] == 0
                    bs_.append(lax.shift_right_logical(key,
                                                       jnp.uint32(21)))
                    vals.append(valid)
                    kss.append(jnp.where(valid, key,
                                         jnp.uint32(_SENTINEL)))
                for hh, (_l, _m, ko_v) in enumerate(halves):
                    for uu in range(_UNROLL):
                        cc = hh * _UNROLL + uu
                        plsc.addupdate_scatter(
                            hists[uu], [plsc.bitcast(bs_[cc], jnp.int32)],
                            ones, mask=vals[cc])
                        ko_v[pl.ds(i + uu * _L, _L)] = plsc.bitcast(
                            kss[cc], jnp.int32)

        pltpu.emit_pipeline(
            block,
            grid=(_G1,),
            in_specs=[pl.BlockSpec((_CH1,), lambda i: (i,)),
                      pl.BlockSpec((_CH1,), lambda i: (i + _G1,)),
                      pl.BlockSpec((_CH1,), lambda i: (i,)),
                      pl.BlockSpec((_CH1,), lambda i: (i + _G1,)),
                      ],
            out_specs=[pl.BlockSpec((_CH1,), lambda i: (i,)),
                       pl.BlockSpec((_CH1,), lambda i: (i,))],
            core_axis_name=("c", "s"),
            dimension_semantics=(pltpu.PARALLEL,),
        )(lp_hbm, lp_hbm, mk_hbm, mk_hbm, ka_hbm, kb_hbm)

        @pl.loop(0, nb, step=_L)
        def _sum(i):
            s = pl.ds(i, _L)
            hists[0][s] = ((hists[0][s] + hists[1][s])
                           + (hists[2][s] + hists[3][s]))

        pltpu.sync_copy(hists[0], out_hbm.at[wid])

    return pl.kernel(
        body, mesh=mesh,
        out_type=[jax.ShapeDtypeStruct((_NW, nb), jnp.int32),
                  jax.ShapeDtypeStruct((_NHALF,), jnp.int32),
                  jax.ShapeDtypeStruct((_NHALF,), jnp.int32)],
        scratch_types=[pltpu.VMEM((nb,), jnp.int32)] * _UNROLL,
        compiler_params=_compiler_params(),
    )


def _sc_pass23(nbits, shift):
    nb = 1 << nbits
    mesh = plsc.VectorSubcoreMesh(core_axis_name="c", subcore_axis_name="s")

    def body(ka_hbm, kb_hbm, pfx_hbm, out_hbm, pfx_v, *hists):
        pltpu.sync_copy(pfx_hbm, pfx_v)
        wid = lax.axis_index("s") * _NC + lax.axis_index("c")

        @pl.loop(0, nb, step=_L)
        def _zero(i):
            for h in hists:
                h[pl.ds(i, _L)] = jnp.zeros((_L,), jnp.int32)

        ones = jnp.ones((_L,), jnp.int32)

        def block(ka0_v, ka1_v, kb0_v, kb1_v):
            pshift = lax.shift_left(pfx_v[...], jnp.int32(nbits))
            krefs = (ka0_v, ka1_v, kb0_v, kb1_v)

            @pl.loop(0, _CH2, step=2 * _L)
            def _(i):
                keys = [plsc.bitcast(k_v[pl.ds(i + uu * _L, _L)],
                                     jnp.uint32)
                        for k_v in krefs for uu in range(2)]
                ds_ = []
                vs_ = []
                for uu in range(8):
                    r = lax.shift_right_logical(keys[uu],
                                                jnp.uint32(shift))
                    d = plsc.bitcast(r, jnp.int32) - pshift
                    ds_.append(d)
                    vs_.append(plsc.bitcast(d, jnp.uint32)
                               < jnp.uint32(nb))
                for uu in range(8):
                    plsc.addupdate_scatter(hists[uu % _UNROLL], [ds_[uu]],
                                           ones, mask=vs_[uu])

        pltpu.emit_pipeline(
            block,
            grid=(_G2,),
            in_specs=[pl.BlockSpec((_CH2,), lambda i: (i,)),
                      pl.BlockSpec((_CH2,), lambda i: (i + _G2,)),
                      pl.BlockSpec((_CH2,), lambda i: (i,)),
                      pl.BlockSpec((_CH2,), lambda i: (i + _G2,))],
            out_specs=[],
            core_axis_name=("c", "s"),
            dimension_semantics=(pltpu.PARALLEL,),
        )(ka_hbm, ka_hbm, kb_hbm, kb_hbm)

        @pl.loop(0, nb, step=_L)
        def _sum(i):
            s = pl.ds(i, _L)
            hists[0][s] = ((hists[0][s] + hists[1][s])
                           + (hists[2][s] + hists[3][s]))

        pltpu.sync_copy(hists[0], out_hbm.at[wid])

    return pl.kernel(
        body, mesh=mesh,
        out_type=jax.ShapeDtypeStruct((_NW, nb), jnp.int32),
        scratch_types=[pltpu.VMEM((_L,), jnp.int32)]
                      + [pltpu.VMEM((nb,), jnp.int32)] * _UNROLL,
        compiler_params=_compiler_params(),
    )


def _find_bucket(h2, kf):
    r = h2.shape[0]
    i0 = lax.broadcasted_iota(jnp.int32, (128, 128), 0)
    i1 = lax.broadcasted_iota(jnp.int32, (128, 128), 1)
    tri = (i0 <= i1).astype(jnp.float32)
    c_row = jnp.dot(h2, tri, preferred_element_type=jnp.float32)
    r0 = lax.broadcasted_iota(jnp.int32, (r, r), 0)
    r1 = lax.broadcasted_iota(jnp.int32, (r, r), 1)
    strict = (r1 < r0).astype(jnp.float32)
    above = jnp.dot(strict, h2, preferred_element_type=jnp.float32)
    offs = jnp.sum(above, axis=1, keepdims=True)
    cum = c_row + offs
    bt = jnp.sum((cum < kf).astype(jnp.int32))
    j0 = lax.broadcasted_iota(jnp.int32, (r, 128), 0)
    j1 = lax.broadcasted_iota(jnp.int32, (r, 128), 1)
    lin = j0 * 128 + j1
    before = jnp.sum(jnp.where(lin < bt, h2, 0.0))
    return bt, before.astype(jnp.int32)


def _merge(ph_ref):
    nb = ph_ref.shape[1]
    h = ph_ref[...].astype(jnp.float32)
    h3 = h.reshape(_NW, nb // 128, 128)
    return jnp.sum(h3, axis=0)


def _pack_meta(k_rem, nv):
    li = lax.broadcasted_iota(jnp.int32, (1, _L), 1)
    return jnp.where(li == 0, k_rem, jnp.where(li == 1, nv, 0))


def _tc_sel1_body(ph_ref, pfx_ref, meta_ref):
    h2 = _merge(ph_ref)
    nvf = jnp.sum(h2)
    k = (nvf * jnp.float32(_POS_BETA)).astype(jnp.int32)
    bt, before = _find_bucket(h2, k.astype(jnp.float32) + 1.0)
    pfx_ref[...] = jnp.full((1, _L), bt, jnp.int32)
    meta_ref[...] = _pack_meta(k - before, nvf.astype(jnp.int32))


def _tc_sel2_body(ph_ref, pfx_in_ref, meta_in_ref, pfx_ref, meta_ref):
    h2 = _merge(ph_ref)
    k1 = meta_in_ref[0, 0]
    nv = meta_in_ref[0, 1]
    b0 = pfx_in_ref[0, 0]
    bt, before = _find_bucket(h2, k1.astype(jnp.float32) + 1.0)
    pfx2 = lax.shift_left(b0, 11) | bt
    pfx_ref[...] = jnp.full((1, _L), pfx2, jnp.int32)
    meta_ref[...] = _pack_meta(k1 - before, nv)


def _tc_final_body(ph_ref, pfx_in_ref, meta_in_ref, bn_ref, ba_ref):
    h2 = _merge(ph_ref)
    k2 = meta_in_ref[0, 0]
    nv = meta_in_ref[0, 1]
    p01 = pfx_in_ref[0, 0]
    bt, _ = _find_bucket(h2, k2.astype(jnp.float32) + 1.0)
    key = lax.shift_left(p01, 10) | bt
    key11 = jnp.full((1, 1), key, jnp.int32)
    ku = lax.bitcast_convert_type(key11, jnp.uint32)
    u = jnp.where(ku >= jnp.uint32(0x80000000),
                  ku ^ jnp.uint32(0x80000000), ~ku)
    val = lax.bitcast_convert_type(u, jnp.float32)
    bn = val / jnp.float32(_NORMALIZER)
    bn = jnp.where(jnp.full((1, 1), nv, jnp.int32) == 0,
                   jnp.float32(jnp.inf), bn)
    bn_ref[...] = bn
    ba_ref[...] = bn - jnp.float32(_MARGIN_TAU)


_sc_cache = {}


def _sc_kernel(tag):
    if tag not in _sc_cache:
        if tag == 1:
            _sc_cache[tag] = _sc_pass1()
        elif tag == 2:
            _sc_cache[tag] = _sc_pass23(11, 10)
        else:
            _sc_cache[tag] = _sc_pass23(10, 0)
    return _sc_cache[tag]


_tc_sel1 = pl.pallas_call(
    _tc_sel1_body,
    out_shape=[jax.ShapeDtypeStruct((1, _L), jnp.int32),
               jax.ShapeDtypeStruct((1, _L), jnp.int32)],
)
_tc_sel2 = pl.pallas_call(
    _tc_sel2_body,
    out_shape=[jax.ShapeDtypeStruct((1, _L), jnp.int32),
               jax.ShapeDtypeStruct((1, _L), jnp.int32)],
)
_tc_final = pl.pallas_call(
    _tc_final_body,
    out_shape=[jax.ShapeDtypeStruct((1, 1), jnp.float32),
               jax.ShapeDtypeStruct((1, 1), jnp.float32)],
)


def kernel(logps, mask):
    lp = jnp.concatenate([logps, jnp.zeros((_PAD,), jnp.float32)])
    mk = jnp.concatenate([mask, jnp.ones((_PAD,), jnp.int32)])
    ph1, ka, kb = _sc_kernel(1)(lp, mk)
    pfx1, meta1 = _tc_sel1(ph1)
    ph2 = _sc_kernel(2)(ka, kb, pfx1.reshape(_L))
    pfx2, meta2 = _tc_sel2(ph2, pfx1, meta1)
    ph3 = _sc_kernel(3)(ka, kb, pfx2.reshape(_L))
    bn, ba = _tc_final(ph3, pfx2, meta2)
    return bn.reshape(()), ba.reshape(())
